# baseline (speedup 1.0000x reference)
import functools

import jax
import jax.numpy as jnp
from jax import lax
from jax.experimental import pallas as pl
from jax.experimental.pallas import tpu as pltpu
from jax.experimental.pallas import tpu_sc as plsc

_B = 16
_TOTAL_UIH = 32768
_NUM_CAND = 128
_DIM = 128
_TOTAL_CAND = _B * _NUM_CAND
_TOTAL_OUT = _TOTAL_UIH + _TOTAL_CAND
_NW = 32
_UIH_W = _TOTAL_UIH // _NW
_CAND_W = _TOTAL_CAND // _NW
_CHUNK = 128
_NCH_U = _UIH_W // _CHUNK
_NCH = _NCH_U + 1
_NBUF = 6
_S = _CHUNK // 16

_mesh = plsc.VectorSubcoreMesh(core_axis_name="c", subcore_axis_name="s")


def _dyn_gather(vec, idx):
    return vec.at[idx].get(mode="promise_in_bounds")


@functools.partial(
    pl.kernel,
    mesh=_mesh,
    out_type=(
        jax.ShapeDtypeStruct((_TOTAL_OUT, _DIM), jnp.float32),
        jax.ShapeDtypeStruct((_B,), jnp.int32),
        jax.ShapeDtypeStruct((_B,), jnp.int32),
        jax.ShapeDtypeStruct((_B,), jnp.int32),
        jax.ShapeDtypeStruct((_TOTAL_UIH,), jnp.int32),
    ),
    compiler_params=pltpu.CompilerParams(needs_layout_passes=False),
    scratch_types=[
        pltpu.VMEM((_UIH_W,), jnp.int32),
        pltpu.VMEM((_CAND_W,), jnp.int32),
        pltpu.VMEM((16,), jnp.int32),
        pltpu.VMEM((_NCH_U, _CHUNK), jnp.int32),
        pltpu.VMEM((1, _CAND_W), jnp.int32),
        pltpu.VMEM((_NBUF, _CHUNK, _DIM), jnp.float32),
        pltpu.VMEM((_CAND_W, _DIM), jnp.float32),
        pltpu.VMEM((16,), jnp.int32),
        pltpu.VMEM((_UIH_W,), jnp.int32),
        pltpu.SemaphoreType.DMA,
        pltpu.SemaphoreType.DMA,
        pltpu.SemaphoreType.DMA,
        pltpu.SemaphoreType.DMA,
        pltpu.SemaphoreType.DMA,
    ],
)
def _sc_kernel(uih_hbm, cand_hbm, inner_hbm, ts_hbm, table_hbm,
               out_hbm, olen_hbm, ulen_hbm, ncand_hbm, ts_out_hbm,
               ids_u, ids_c, off_v, dest_v, dest_c, rows_v, rows_c,
               small_v, ts_v, sem_s, sem_g, sem_w, sem_t, sem_c):
    wid = lax.axis_index("s") * 2 + lax.axis_index("c")
    ubase = wid * _UIH_W
    cbase = wid * _CAND_W

    st_u = pltpu.async_copy(uih_hbm.at[pl.ds(ubase, _UIH_W)], ids_u, sem_s)
    st_c = pltpu.async_copy(cand_hbm.at[pl.ds(cbase, _CAND_W)], ids_c, sem_s)
    st_t = pltpu.async_copy(ts_hbm.at[pl.ds(ubase, _UIH_W)], ts_v, sem_t)
    pltpu.sync_copy(inner_hbm, off_v.at[pl.ds(0, _B - 1)])

    iota = lax.iota(jnp.int32, 16)
    v = jnp.where(iota < _B - 1, off_v[...], _TOTAL_UIH)

    def rank16(pos):
        seg = jnp.zeros((16,), jnp.int32)
        for stp in (8, 4, 2, 1):
            probe = _dyn_gather(v, seg + (stp - 1))
            seg = jnp.where(pos >= probe, seg + stp, seg)
        return seg

    def dest_chunk(j):
        def body_u(t, carry):
            k = ubase + j * _CHUNK + t * 16 + iota
            dest_v[j, pl.ds(t * 16, 16)] = k + _NUM_CAND * rank16(k)
            return carry
        lax.fori_loop(0, _S, body_u, 0)

    def dest_cand(t, carry):
        cf = cbase + t * 16 + iota
        i = lax.div(cf, _NUM_CAND)
        c = cf - i * _NUM_CAND
        dest_c[0, pl.ds(t * 16, 16)] = _dyn_gather(v, i) + _NUM_CAND * i + c
        return carry

    @pl.when(wid == 0)
    def _():
        u_lo = jnp.where(iota == 0, 0,
                         _dyn_gather(v, jnp.maximum(iota - 1, 0)))
        ulen = v - u_lo
        small_v[...] = ulen
        pltpu.sync_copy(small_v, ulen_hbm)
        small_v[...] = ulen + _NUM_CAND
        pltpu.sync_copy(small_v, olen_hbm)
        small_v[...] = jnp.full((16,), _NUM_CAND, jnp.int32)
        pltpu.sync_copy(small_v, ncand_hbm)

    lax.fori_loop(0, _CAND_W // 16, dest_cand, 0)
    dest_chunk(0)
    dest_chunk(1)
    st_u.wait()
    st_c.wait()
    g_c = pltpu.async_copy(table_hbm.at[ids_c], rows_c, sem_c)

    gathers = [None] * _NCH_U
    writes = [None] * _NCH_U
    for j in range(_NCH_U):
        if j >= _NBUF:
            writes[j - _NBUF].wait()
        gathers[j] = pltpu.async_copy(
            table_hbm.at[ids_u.at[pl.ds(j * _CHUNK, _CHUNK)]],
            rows_v.at[j % _NBUF], sem_g)
        if j + 2 < _NCH_U:
            dest_chunk(j + 2)
        if j > 0:
            gathers[j - 1].wait()
            writes[j - 1] = pltpu.async_copy(
                rows_v.at[(j - 1) % _NBUF],
                out_hbm.at[dest_v.at[j - 1]], sem_w)
    gathers[_NCH_U - 1].wait()
    writes[_NCH_U - 1] = pltpu.async_copy(
        rows_v.at[(_NCH_U - 1) % _NBUF],
        out_hbm.at[dest_v.at[_NCH_U - 1]], sem_w)

    g_c.wait()
    w_c = pltpu.async_copy(rows_c, out_hbm.at[dest_c.at[0]], sem_c)
    st_t.wait()
    pltpu.sync_copy(ts_v, ts_out_hbm.at[pl.ds(ubase, _UIH_W)])
    for j in range(_NCH_U - _NBUF, _NCH_U):
        writes[j].wait()
    w_c.wait()


def kernel(uih_values, uih_inner_offsets, cand_values, uih_timestamps, table):
    emb, out_lengths, uih_seq_lengths, num_candidates, ts = _sc_kernel(
        uih_values.astype(jnp.int32),
        cand_values.astype(jnp.int32),
        uih_inner_offsets.astype(jnp.int32),
        uih_timestamps.astype(jnp.int32),
        table,
    )
    return (emb, out_lengths, ts, uih_seq_lengths, num_candidates)


# speedup vs baseline: 1.0066x; 1.0066x over previous
import functools

import jax
import jax.numpy as jnp
from jax import lax
from jax.experimental import pallas as pl
from jax.experimental.pallas import tpu as pltpu
from jax.experimental.pallas import tpu_sc as plsc

_B = 16
_TOTAL_UIH = 32768
_NUM_CAND = 128
_DIM = 128
_TOTAL_CAND = _B * _NUM_CAND
_TOTAL_OUT = _TOTAL_UIH + _TOTAL_CAND
_NW = 32
_UIH_W = _TOTAL_UIH // _NW
_CAND_W = _TOTAL_CAND // _NW
_CHUNK = 128
_NCH_U = _UIH_W // _CHUNK
_NCH = _NCH_U + 1
_NBUF = 4
_S = _CHUNK // 16

_mesh = plsc.VectorSubcoreMesh(core_axis_name="c", subcore_axis_name="s")


def _dyn_gather(vec, idx):
    return vec.at[idx].get(mode="promise_in_bounds")


@functools.partial(
    pl.kernel,
    mesh=_mesh,
    out_type=(
        jax.ShapeDtypeStruct((_TOTAL_OUT, _DIM), jnp.float32),
        jax.ShapeDtypeStruct((_B,), jnp.int32),
        jax.ShapeDtypeStruct((_B,), jnp.int32),
        jax.ShapeDtypeStruct((_B,), jnp.int32),
        jax.ShapeDtypeStruct((_TOTAL_UIH,), jnp.int32),
    ),
    compiler_params=pltpu.CompilerParams(needs_layout_passes=False),
    scratch_types=[
        pltpu.VMEM((_UIH_W,), jnp.int32),
        pltpu.VMEM((_CAND_W,), jnp.int32),
        pltpu.VMEM((16,), jnp.int32),
        pltpu.VMEM((_NCH_U, _CHUNK), jnp.int32),
        pltpu.VMEM((1, _CAND_W), jnp.int32),
        pltpu.VMEM((_NBUF, _CHUNK, _DIM), jnp.float32),
        pltpu.VMEM((_CAND_W, _DIM), jnp.float32),
        pltpu.VMEM((16,), jnp.int32),
        pltpu.VMEM((_UIH_W,), jnp.int32),
        pltpu.SemaphoreType.DMA,
        pltpu.SemaphoreType.DMA,
        pltpu.SemaphoreType.DMA,
        pltpu.SemaphoreType.DMA,
        pltpu.SemaphoreType.DMA,
    ],
)
def _sc_kernel(uih_hbm, cand_hbm, inner_hbm, ts_hbm, table_hbm,
               out_hbm, olen_hbm, ulen_hbm, ncand_hbm, ts_out_hbm,
               ids_u, ids_c, off_v, dest_v, dest_c, rows_v, rows_c,
               small_v, ts_v, sem_s, sem_g, sem_w, sem_t, sem_c):
    wid = lax.axis_index("s") * 2 + lax.axis_index("c")
    ubase = wid * _UIH_W
    cbase = wid * _CAND_W

    st_u = pltpu.async_copy(uih_hbm.at[pl.ds(ubase, _UIH_W)], ids_u, sem_s)
    st_c = pltpu.async_copy(cand_hbm.at[pl.ds(cbase, _CAND_W)], ids_c, sem_s)
    st_t = pltpu.async_copy(ts_hbm.at[pl.ds(ubase, _UIH_W)], ts_v, sem_t)
    pltpu.sync_copy(inner_hbm, off_v.at[pl.ds(0, _B - 1)])

    iota = lax.iota(jnp.int32, 16)
    v = jnp.where(iota < _B - 1, off_v[...], _TOTAL_UIH)

    def rank16(pos):
        seg = jnp.zeros((16,), jnp.int32)
        for stp in (8, 4, 2, 1):
            probe = _dyn_gather(v, seg + (stp - 1))
            seg = jnp.where(pos >= probe, seg + stp, seg)
        return seg

    def dest_chunk(j):
        def body_u(t, carry):
            k = ubase + j * _CHUNK + t * 16 + iota
            dest_v[j, pl.ds(t * 16, 16)] = k + _NUM_CAND * rank16(k)
            return carry
        lax.fori_loop(0, _S, body_u, 0)

    def dest_cand(t, carry):
        cf = cbase + t * 16 + iota
        i = lax.div(cf, _NUM_CAND)
        c = cf - i * _NUM_CAND
        dest_c[0, pl.ds(t * 16, 16)] = _dyn_gather(v, i) + _NUM_CAND * i + c
        return carry

    def small_outputs():
        u_lo = jnp.where(iota == 0, 0,
                         _dyn_gather(v, jnp.maximum(iota - 1, 0)))
        ulen = v - u_lo
        small_v[...] = ulen
        pltpu.sync_copy(small_v, ulen_hbm)
        small_v[...] = ulen + _NUM_CAND
        pltpu.sync_copy(small_v, olen_hbm)
        small_v[...] = jnp.full((16,), _NUM_CAND, jnp.int32)
        pltpu.sync_copy(small_v, ncand_hbm)

    lax.fori_loop(0, _CAND_W // 16, dest_cand, 0)
    dest_chunk(0)
    dest_chunk(1)
    st_u.wait()
    st_c.wait()
    g_c = pltpu.async_copy(table_hbm.at[ids_c], rows_c, sem_c)

    gathers = [None] * _NCH_U
    writes = [None] * _NCH_U
    for j in range(_NCH_U):
        if j >= _NBUF:
            writes[j - _NBUF].wait()
        gathers[j] = pltpu.async_copy(
            table_hbm.at[ids_u.at[pl.ds(j * _CHUNK, _CHUNK)]],
            rows_v.at[j % _NBUF], sem_g)
        if j + 2 < _NCH_U:
            dest_chunk(j + 2)
        if j > 0:
            gathers[j - 1].wait()
            writes[j - 1] = pltpu.async_copy(
                rows_v.at[(j - 1) % _NBUF],
                out_hbm.at[dest_v.at[j - 1]], sem_w)
    gathers[_NCH_U - 1].wait()
    writes[_NCH_U - 1] = pltpu.async_copy(
        rows_v.at[(_NCH_U - 1) % _NBUF],
        out_hbm.at[dest_v.at[_NCH_U - 1]], sem_w)

    g_c.wait()
    w_c = pltpu.async_copy(rows_c, out_hbm.at[dest_c.at[0]], sem_c)
    pl.when(wid == 0)(small_outputs)
    st_t.wait()
    pltpu.sync_copy(ts_v, ts_out_hbm.at[pl.ds(ubase, _UIH_W)])
    for j in range(_NCH_U - _NBUF, _NCH_U):
        writes[j].wait()
    w_c.wait()


def kernel(uih_values, uih_inner_offsets, cand_values, uih_timestamps, table):
    emb, out_lengths, uih_seq_lengths, num_candidates, ts = _sc_kernel(
        uih_values.astype(jnp.int32),
        cand_values.astype(jnp.int32),
        uih_inner_offsets.astype(jnp.int32),
        uih_timestamps.astype(jnp.int32),
        table,
    )
    return (emb, out_lengths, ts, uih_seq_lengths, num_candidates)
